# baseline (speedup 1.0000x reference)
import functools

import jax
import jax.numpy as jnp
from jax import lax
from jax.experimental import pallas as pl
from jax.experimental.pallas import tpu as pltpu
from jax.experimental.pallas import tpu_sc as plsc

H = 16
D = 64
NG_DIM = 8
UNI_V = 1024
NG_V = 768 * 256
B = 4
N = 2048

NUM_TOKENS = B * N
NUM_WORKERS = 32
TOK_PER_W = NUM_TOKENS // NUM_WORKERS
ELEMS_PER_W = TOK_PER_W * H * NG_DIM
GCHUNK = 16384
NUM_CHUNKS = ELEMS_PER_W // GCHUNK
TOK_PER_CHUNK = GCHUNK // (H * NG_DIM)
ROW_TILES = NG_V // 128


def _primes_above(start, count):
    out = []
    x = start
    while len(out) < count:
        is_p = x > 1
        for p in range(2, int(x ** 0.5) + 1):
            if x % p == 0:
                is_p = False
                break
        if is_p:
            out.append(x)
        x += 1
    return out


_PRIMES = _primes_above(NG_V + 1, H)


def _sc_gather_kernel(ids_hbm, primes_hbm, table_hbm, out_hbm,
                      ids_v, prev0_v, primes_v, idx_v, rows_v, sem0, sem1):
    wid = lax.axis_index("s") * 2 + lax.axis_index("c")
    base_tok = wid * TOK_PER_W
    base_row = base_tok * H

    pltpu.sync_copy(ids_hbm.at[pl.ds(base_row, TOK_PER_W * H)], ids_v)
    pltpu.sync_copy(primes_hbm, primes_v)
    is_boundary = (base_tok % N) == 0
    safe_off = lax.select(is_boundary, 0, base_row - H)
    pltpu.sync_copy(ids_hbm.at[pl.ds(safe_off, H)], prev0_v)

    primes = primes_v[...]
    hid = lax.iota(jnp.int32, H)
    heads = hid + 1
    lane8 = hid * NG_DIM

    keep = lax.select(is_boundary, jnp.int32(0), jnp.int32(1))
    prev = prev0_v[...] * keep

    def body(k, prev):
        cur = ids_v[pl.ds(k * H, H)]
        ngram = cur + prev * UNI_V
        r = (ngram * heads + heads) % primes % NG_V
        base = (hid * ROW_TILES + (r >> 7)) * (NG_DIM * 128) + (r & 127)
        row = jnp.full((H,), k // TOK_PER_CHUNK, jnp.int32)
        col0 = (k % TOK_PER_CHUNK) * (H * NG_DIM)
        for d in range(NG_DIM):
            plsc.store_scatter(idx_v, [row, col0 + lane8 + d], base + d * 128)
        return cur

    copies = []
    for j in range(NUM_CHUNKS):
        prev = lax.fori_loop(j * TOK_PER_CHUNK, (j + 1) * TOK_PER_CHUNK,
                             body, prev, unroll=4)
        sem = sem0 if j % 2 == 0 else sem1
        c = pltpu.make_async_copy(
            table_hbm.at[idx_v.at[j]], rows_v.at[j], sem)
        c.start()
        copies.append(c)
    for c in copies:
        c.wait()

    pltpu.sync_copy(rows_v, out_hbm.at[wid])


def _sc_gather(ids_flat, primes, table_flat):
    mesh = plsc.VectorSubcoreMesh(core_axis_name="c", subcore_axis_name="s")
    kern = pl.kernel(
        _sc_gather_kernel,
        out_type=jax.ShapeDtypeStruct((NUM_WORKERS, NUM_CHUNKS, GCHUNK),
                                      jnp.float32),
        mesh=mesh,
        scratch_types=[
            pltpu.VMEM((TOK_PER_W * H,), jnp.int32),
            pltpu.VMEM((H,), jnp.int32),
            pltpu.VMEM((H,), jnp.int32),
            pltpu.VMEM((NUM_CHUNKS, GCHUNK), jnp.int32),
            pltpu.VMEM((NUM_CHUNKS, GCHUNK), jnp.float32),
            pltpu.SemaphoreType.DMA,
            pltpu.SemaphoreType.DMA,
        ],
        compiler_params=pltpu.CompilerParams(use_tc_tiling_on_sc=False,
                                             needs_layout_passes=False),
    )
    return kern(ids_flat, primes, table_flat)


def _tc_ln_kernel(e_ref, g_ref, m8_ref, m64_ref, p2w_ref, iwz_ref, bias_ref,
                  out_ref):
    eps = 1e-5
    g = g_ref[...]
    m8 = m8_ref[...]
    mu8 = jax.lax.dot(g, m8)
    var8 = jax.lax.dot(g * g, m8) - mu8 * mu8
    norm = (g - mu8) * lax.rsqrt(var8 + eps)
    lng = jax.lax.dot(norm, p2w_ref[...])

    m64 = m64_ref[...]
    iwz = iwz_ref[...]
    bias = bias_ref[...]
    for p in range(H // 2):
        sl = slice(p * 2 * D, (p + 1) * 2 * D)
        ep = e_ref[:, sl]
        mu = jax.lax.dot(ep, m64)
        var = jax.lax.dot(ep * ep, m64) - mu * mu
        lne = (ep - mu) * lax.rsqrt(var + eps) * iwz[:, sl]
        out_ref[:, sl] = lne + (lng[:, sl] + bias[:, sl])


def _tc_ln(e2, g2, m8, m64, p2w, iwz, bias, block_t=1024):
    nch = NUM_TOKENS // block_t
    return pl.pallas_call(
        _tc_ln_kernel,
        grid=(nch,),
        in_specs=[
            pl.BlockSpec((block_t, H * D), lambda i: (i, 0)),
            pl.BlockSpec((block_t, H * NG_DIM), lambda i: (i, 0)),
            pl.BlockSpec((H * NG_DIM, H * NG_DIM), lambda i: (0, 0)),
            pl.BlockSpec((2 * D, 2 * D), lambda i: (0, 0)),
            pl.BlockSpec((H * NG_DIM, H * D), lambda i: (0, 0)),
            pl.BlockSpec((1, H * D), lambda i: (0, 0)),
            pl.BlockSpec((1, H * D), lambda i: (0, 0)),
        ],
        out_specs=pl.BlockSpec((block_t, H * D), lambda i: (i, 0)),
        out_shape=jax.ShapeDtypeStruct((NUM_TOKENS, H * D), jnp.float32),
    )(e2, g2, m8, m64, p2w, iwz, bias)


@jax.jit
def kernel(embeds, cluster_ids, emb_tables, in_norm_w, in_norm_b,
           ng_norm_w, ng_norm_b):
    b, n, _ = embeds.shape
    ids_flat = cluster_ids.reshape(-1)
    table_flat = (emb_tables
                  .transpose(0, 2, 1)
                  .reshape(H, NG_DIM, ROW_TILES, 128)
                  .transpose(0, 2, 1, 3)
                  .reshape(H * NG_V * NG_DIM))
    primes = jnp.asarray(_PRIMES, dtype=jnp.int32)

    ng_flat = _sc_gather(ids_flat, primes, table_flat)

    e2 = embeds.reshape(NUM_TOKENS, H * D)
    g2 = ng_flat.reshape(NUM_TOKENS, H * NG_DIM)

    nl = H * NG_DIM
    gi = jnp.arange(nl)
    m8 = jnp.where(gi[:, None] // NG_DIM == gi[None, :] // NG_DIM,
                   1.0 / NG_DIM, 0.0).astype(jnp.float32)
    di = jnp.arange(2 * D)
    m64 = jnp.where(di[:, None] // D == di[None, :] // D,
                    1.0 / D, 0.0).astype(jnp.float32)
    tgt = (gi // NG_DIM) * D + (D - NG_DIM) + gi % NG_DIM
    oi = jnp.arange(H * D)
    p2w = jnp.where(oi[None, :] == tgt[:, None],
                    ng_norm_w.reshape(nl, 1), 0.0).astype(jnp.float32)
    lane_c = oi % D
    dense_mask = (lane_c < D - NG_DIM).astype(jnp.float32)
    iwz = (in_norm_w.reshape(1, H * D) * dense_mask[None])
    gb_full = jnp.where(oi[None, :] == tgt[:, None],
                        ng_norm_b.reshape(nl, 1), 0.0).sum(0)
    bias = (in_norm_b.reshape(1, H * D) * dense_mask[None]
            + gb_full[None].astype(jnp.float32))

    out = _tc_ln(e2, g2, m8, m64, p2w, iwz, bias)
    return out.reshape(b, n, H * D)


# speedup vs baseline: 1.0455x; 1.0455x over previous
import functools

import jax
import jax.numpy as jnp
from jax import lax
from jax.experimental import pallas as pl
from jax.experimental.pallas import tpu as pltpu
from jax.experimental.pallas import tpu_sc as plsc

H = 16
D = 64
NG_DIM = 8
UNI_V = 1024
NG_V = 768 * 256
B = 4
N = 2048

NUM_TOKENS = B * N
NUM_WORKERS = 32
TOK_PER_W = NUM_TOKENS // NUM_WORKERS
ELEMS_PER_W = TOK_PER_W * H * NG_DIM
GCHUNK = 8192
NUM_CHUNKS = ELEMS_PER_W // GCHUNK
TOK_PER_CHUNK = GCHUNK // (H * NG_DIM)
ROW_TILES = NG_V // 128


def _primes_above(start, count):
    out = []
    x = start
    while len(out) < count:
        is_p = x > 1
        for p in range(2, int(x ** 0.5) + 1):
            if x % p == 0:
                is_p = False
                break
        if is_p:
            out.append(x)
        x += 1
    return out


_PRIMES = _primes_above(NG_V + 1, H)


def _sc_gather_kernel(ids_hbm, primes_hbm, table_hbm, out_hbm,
                      ids_v, prev0_v, primes_v, idx_v, rows_v, sem0, sem1):
    wid = lax.axis_index("s") * 2 + lax.axis_index("c")
    base_tok = wid * TOK_PER_W
    base_row = base_tok * H

    pltpu.sync_copy(ids_hbm.at[pl.ds(base_row, TOK_PER_W * H)], ids_v)
    pltpu.sync_copy(primes_hbm, primes_v)
    is_boundary = (base_tok % N) == 0
    safe_off = lax.select(is_boundary, 0, base_row - H)
    pltpu.sync_copy(ids_hbm.at[pl.ds(safe_off, H)], prev0_v)

    primes = primes_v[...]
    hid = lax.iota(jnp.int32, H)
    heads = hid + 1
    lane8 = hid * NG_DIM

    keep = lax.select(is_boundary, jnp.int32(0), jnp.int32(1))
    prev = prev0_v[...] * keep

    def body(k, prev):
        cur = ids_v[pl.ds(k * H, H)]
        ngram = cur + prev * UNI_V
        r = (ngram * heads + heads) % primes % NG_V
        base = (hid * ROW_TILES + (r >> 7)) * (NG_DIM * 128) + (r & 127)
        row = jnp.full((H,), k // TOK_PER_CHUNK, jnp.int32)
        col0 = (k % TOK_PER_CHUNK) * (H * NG_DIM)
        for d in range(NG_DIM):
            plsc.store_scatter(idx_v, [row, col0 + lane8 + d], base + d * 128)
        return cur

    copies = []
    for j in range(NUM_CHUNKS):
        prev = lax.fori_loop(j * TOK_PER_CHUNK, (j + 1) * TOK_PER_CHUNK,
                             body, prev, unroll=4)
        sem = sem0 if j % 2 == 0 else sem1
        c = pltpu.make_async_copy(
            table_hbm.at[idx_v.at[j]], rows_v.at[j], sem)
        c.start()
        copies.append(c)
    for c in copies:
        c.wait()

    pltpu.sync_copy(rows_v, out_hbm.at[wid])


def _sc_gather(ids_flat, primes, table_flat):
    mesh = plsc.VectorSubcoreMesh(core_axis_name="c", subcore_axis_name="s")
    kern = pl.kernel(
        _sc_gather_kernel,
        out_type=jax.ShapeDtypeStruct((NUM_WORKERS, NUM_CHUNKS, GCHUNK),
                                      jnp.float32),
        mesh=mesh,
        scratch_types=[
            pltpu.VMEM((TOK_PER_W * H,), jnp.int32),
            pltpu.VMEM((H,), jnp.int32),
            pltpu.VMEM((H,), jnp.int32),
            pltpu.VMEM((NUM_CHUNKS, GCHUNK), jnp.int32),
            pltpu.VMEM((NUM_CHUNKS, GCHUNK), jnp.float32),
            pltpu.SemaphoreType.DMA,
            pltpu.SemaphoreType.DMA,
        ],
        compiler_params=pltpu.CompilerParams(use_tc_tiling_on_sc=False,
                                             needs_layout_passes=False),
    )
    return kern(ids_flat, primes, table_flat)


def _tc_ln_kernel(e_ref, g_ref, m8_ref, m64_ref, p2w_ref, iwz_ref, bias_ref,
                  out_ref):
    eps = 1e-5
    g = g_ref[...]
    m8 = m8_ref[...]
    mu8 = jax.lax.dot(g, m8)
    var8 = jax.lax.dot(g * g, m8) - mu8 * mu8
    norm = (g - mu8) * lax.rsqrt(var8 + eps)
    lng = jax.lax.dot(norm, p2w_ref[...])

    m64 = m64_ref[...]
    iwz = iwz_ref[...]
    bias = bias_ref[...]
    for p in range(H // 2):
        sl = slice(p * 2 * D, (p + 1) * 2 * D)
        ep = e_ref[:, sl]
        mu = jax.lax.dot(ep, m64)
        var = jax.lax.dot(ep * ep, m64) - mu * mu
        lne = (ep - mu) * lax.rsqrt(var + eps) * iwz[:, sl]
        out_ref[:, sl] = lne + (lng[:, sl] + bias[:, sl])


def _tc_ln(e2, g2, m8, m64, p2w, iwz, bias, block_t=1024):
    nch = NUM_TOKENS // block_t
    return pl.pallas_call(
        _tc_ln_kernel,
        grid=(nch,),
        in_specs=[
            pl.BlockSpec((block_t, H * D), lambda i: (i, 0)),
            pl.BlockSpec((block_t, H * NG_DIM), lambda i: (i, 0)),
            pl.BlockSpec((H * NG_DIM, H * NG_DIM), lambda i: (0, 0)),
            pl.BlockSpec((2 * D, 2 * D), lambda i: (0, 0)),
            pl.BlockSpec((H * NG_DIM, H * D), lambda i: (0, 0)),
            pl.BlockSpec((1, H * D), lambda i: (0, 0)),
            pl.BlockSpec((1, H * D), lambda i: (0, 0)),
        ],
        out_specs=pl.BlockSpec((block_t, H * D), lambda i: (i, 0)),
        out_shape=jax.ShapeDtypeStruct((NUM_TOKENS, H * D), jnp.float32),
    )(e2, g2, m8, m64, p2w, iwz, bias)


@jax.jit
def kernel(embeds, cluster_ids, emb_tables, in_norm_w, in_norm_b,
           ng_norm_w, ng_norm_b):
    b, n, _ = embeds.shape
    ids_flat = cluster_ids.reshape(-1)
    table_flat = (emb_tables
                  .transpose(0, 2, 1)
                  .reshape(H, NG_DIM, ROW_TILES, 128)
                  .transpose(0, 2, 1, 3)
                  .reshape(H * NG_V * NG_DIM))
    primes = jnp.asarray(_PRIMES, dtype=jnp.int32)

    ng_flat = _sc_gather(ids_flat, primes, table_flat)

    e2 = embeds.reshape(NUM_TOKENS, H * D)
    g2 = ng_flat.reshape(NUM_TOKENS, H * NG_DIM)

    nl = H * NG_DIM
    gi = jnp.arange(nl)
    m8 = jnp.where(gi[:, None] // NG_DIM == gi[None, :] // NG_DIM,
                   1.0 / NG_DIM, 0.0).astype(jnp.float32)
    di = jnp.arange(2 * D)
    m64 = jnp.where(di[:, None] // D == di[None, :] // D,
                    1.0 / D, 0.0).astype(jnp.float32)
    tgt = (gi // NG_DIM) * D + (D - NG_DIM) + gi % NG_DIM
    oi = jnp.arange(H * D)
    p2w = jnp.where(oi[None, :] == tgt[:, None],
                    ng_norm_w.reshape(nl, 1), 0.0).astype(jnp.float32)
    lane_c = oi % D
    dense_mask = (lane_c < D - NG_DIM).astype(jnp.float32)
    iwz = (in_norm_w.reshape(1, H * D) * dense_mask[None])
    gb_full = jnp.where(oi[None, :] == tgt[:, None],
                        ng_norm_b.reshape(nl, 1), 0.0).sum(0)
    bias = (in_norm_b.reshape(1, H * D) * dense_mask[None]
            + gb_full[None].astype(jnp.float32))

    out = _tc_ln(e2, g2, m8, m64, p2w, iwz, bias)
    return out.reshape(b, n, H * D)
